# Initial kernel scaffold; baseline (speedup 1.0000x reference)
import jax
import jax.numpy as jnp
from jax.experimental import pallas as pl


def kernel(x, pe):
    raise NotImplementedError("write your pallas kernel here")


# speedup vs baseline: 2.1290x; 2.1290x over previous
import functools

import jax
import jax.numpy as jnp
from jax import lax
from jax.experimental import pallas as pl
from jax.experimental.pallas import tpu as pltpu
from jax.experimental.pallas import tpu_sc as plsc


def _gather_kernel(n_total, d_model, n_workers, b_per_w, chunk, n_chunks):
    mesh = plsc.VectorSubcoreMesh(core_axis_name="c", subcore_axis_name="s")

    @functools.partial(
        pl.kernel,
        mesh=mesh,
        out_type=jax.ShapeDtypeStruct((n_total, d_model), jnp.float32),
        scratch_types=[
            pltpu.VMEM((n_chunks, chunk), jnp.int32),
            pltpu.VMEM((chunk, d_model), jnp.float32),
            pltpu.SemaphoreType.DMA,
        ],
    )
    def k(table_hbm, idx_hbm, out_hbm, idx_v, rows_v, sem):
        nc = plsc.get_sparse_core_info().num_cores
        wid = lax.axis_index("s") * nc + lax.axis_index("c")
        base = wid * b_per_w
        pltpu.sync_copy(idx_hbm.at[wid], idx_v)
        for c in range(n_chunks):
            pltpu.async_copy(table_hbm.at[idx_v.at[c]], rows_v, sem).wait()
            pltpu.sync_copy(rows_v, out_hbm.at[pl.ds(base + c * chunk, chunk)])

    return k


def kernel(x, pe):
    b, s = x.shape
    v, d = pe.shape
    n = b * s
    info = plsc.get_sparse_core_info()
    nw = info.num_cores * info.num_subcores
    b_per_w = n // nw
    chunk = 64
    n_chunks = b_per_w // chunk
    idx3 = x.astype(jnp.int32).reshape(nw, n_chunks, chunk)
    k = _gather_kernel(n, d, nw, b_per_w, chunk, n_chunks)
    out = k(pe, idx3)
    return out.reshape(b, s, d)


# speedup vs baseline: 2.2953x; 1.0781x over previous
import functools

import jax
import jax.numpy as jnp
from jax import lax
from jax.experimental import pallas as pl
from jax.experimental.pallas import tpu as pltpu
from jax.experimental.pallas import tpu_sc as plsc

_NBUF = 3


def _gather_kernel(n_total, d_model, b_per_w, chunk, n_chunks):
    mesh = plsc.VectorSubcoreMesh(core_axis_name="c", subcore_axis_name="s")

    @functools.partial(
        pl.kernel,
        mesh=mesh,
        out_type=jax.ShapeDtypeStruct((n_total, d_model), jnp.float32),
        scratch_types=[
            pltpu.VMEM((n_chunks, chunk), jnp.int32),
            pltpu.VMEM((_NBUF, chunk, d_model), jnp.float32),
            pltpu.SemaphoreType.DMA((_NBUF,)),
            pltpu.SemaphoreType.DMA((_NBUF,)),
        ],
    )
    def k(table_hbm, idx_hbm, out_hbm, idx_v, rows_v, gsem, ssem):
        nc = plsc.get_sparse_core_info().num_cores
        wid = lax.axis_index("s") * nc + lax.axis_index("c")
        base = wid * b_per_w
        pltpu.sync_copy(idx_hbm.at[wid], idx_v)

        gathers = [None] * n_chunks
        scatters = [None] * n_chunks

        def start_gather(c):
            b = c % _NBUF
            gathers[c] = pltpu.make_async_copy(
                table_hbm.at[idx_v.at[c]], rows_v.at[b], gsem.at[b]
            )
            gathers[c].start()

        def start_scatter(c):
            b = c % _NBUF
            gathers[c].wait()
            scatters[c] = pltpu.make_async_copy(
                rows_v.at[b],
                out_hbm.at[pl.ds(base + c * chunk, chunk)],
                ssem.at[b],
            )
            scatters[c].start()

        for c in range(n_chunks):
            if c >= _NBUF:
                scatters[c - _NBUF].wait()
            start_gather(c)
            if c >= 1:
                start_scatter(c - 1)
        start_scatter(n_chunks - 1)
        for c in range(n_chunks - _NBUF, n_chunks):
            scatters[c].wait()

    return k


def kernel(x, pe):
    b, s = x.shape
    v, d = pe.shape
    n = b * s
    info = plsc.get_sparse_core_info()
    nw = info.num_cores * info.num_subcores
    b_per_w = n // nw
    chunk = 32
    n_chunks = b_per_w // chunk
    idx3 = x.astype(jnp.int32).reshape(nw, n_chunks, chunk)
    k = _gather_kernel(n, d, b_per_w, chunk, n_chunks)
    out = k(pe, idx3)
    return out.reshape(b, s, d)


# speedup vs baseline: 2.3135x; 1.0079x over previous
import functools

import jax
import jax.numpy as jnp
from jax import lax
from jax.experimental import pallas as pl
from jax.experimental.pallas import tpu as pltpu
from jax.experimental.pallas import tpu_sc as plsc

_NBUF = 3


def _gather_kernel(n_total, d_model, b_per_w, chunk, n_chunks):
    mesh = plsc.VectorSubcoreMesh(core_axis_name="c", subcore_axis_name="s")

    @functools.partial(
        pl.kernel,
        mesh=mesh,
        out_type=jax.ShapeDtypeStruct((n_total, d_model), jnp.float32),
        scratch_types=[
            pltpu.VMEM((n_chunks, chunk), jnp.int32),
            pltpu.VMEM((_NBUF, chunk, d_model), jnp.float32),
            pltpu.SemaphoreType.DMA((_NBUF,)),
            pltpu.SemaphoreType.DMA((_NBUF,)),
        ],
    )
    def k(table_hbm, idx_hbm, out_hbm, idx_v, rows_v, gsem, ssem):
        nc = plsc.get_sparse_core_info().num_cores
        wid = lax.axis_index("s") * nc + lax.axis_index("c")
        base = wid * b_per_w
        pltpu.sync_copy(idx_hbm.at[wid], idx_v)

        gathers = [None] * n_chunks
        scatters = [None] * n_chunks

        def start_gather(c):
            b = c % _NBUF
            gathers[c] = pltpu.make_async_copy(
                table_hbm.at[idx_v.at[c]], rows_v.at[b], gsem.at[b]
            )
            gathers[c].start()

        def start_scatter(c):
            b = c % _NBUF
            gathers[c].wait()
            scatters[c] = pltpu.make_async_copy(
                rows_v.at[b],
                out_hbm.at[pl.ds(base + c * chunk, chunk)],
                ssem.at[b],
            )
            scatters[c].start()

        for c in range(_NBUF):
            start_gather(c)
        for c in range(n_chunks):
            start_scatter(c)
            if c + _NBUF < n_chunks:
                scatters[c].wait()
                start_gather(c + _NBUF)
        for c in range(n_chunks - _NBUF, n_chunks):
            scatters[c].wait()

    return k


def kernel(x, pe):
    b, s = x.shape
    v, d = pe.shape
    n = b * s
    info = plsc.get_sparse_core_info()
    nw = info.num_cores * info.num_subcores
    b_per_w = n // nw
    chunk = 32
    n_chunks = b_per_w // chunk
    idx3 = x.astype(jnp.int32).reshape(nw, n_chunks, chunk)
    k = _gather_kernel(n, d, b_per_w, chunk, n_chunks)
    out = k(pe, idx3)
    return out.reshape(b, s, d)


# speedup vs baseline: 2.3622x; 1.0210x over previous
import functools

import jax
import jax.numpy as jnp
from jax import lax
from jax.experimental import pallas as pl
from jax.experimental.pallas import tpu as pltpu
from jax.experimental.pallas import tpu_sc as plsc

_NBUF = 4


def _gather_kernel(n_total, d_model, b_per_w, chunk, n_chunks):
    mesh = plsc.VectorSubcoreMesh(core_axis_name="c", subcore_axis_name="s")

    @functools.partial(
        pl.kernel,
        mesh=mesh,
        out_type=jax.ShapeDtypeStruct((n_total, d_model), jnp.float32),
        scratch_types=[
            pltpu.VMEM((n_chunks, chunk), jnp.int32),
            pltpu.VMEM((_NBUF, chunk, d_model), jnp.float32),
            pltpu.SemaphoreType.DMA((_NBUF,)),
            pltpu.SemaphoreType.DMA((_NBUF,)),
        ],
    )
    def k(table_hbm, idx_hbm, out_hbm, idx_v, rows_v, gsem, ssem):
        nc = plsc.get_sparse_core_info().num_cores
        wid = lax.axis_index("s") * nc + lax.axis_index("c")
        base = wid * b_per_w
        pltpu.sync_copy(idx_hbm.at[wid], idx_v)

        def gather(c, b):
            cp = pltpu.make_async_copy(
                table_hbm.at[idx_v.at[c]], rows_v.at[b], gsem.at[b]
            )
            cp.start()
            return cp

        def scatter(c, b):
            pltpu.make_async_copy(
                table_hbm.at[idx_v.at[c]], rows_v.at[b], gsem.at[b]
            ).wait()
            cp = pltpu.make_async_copy(
                rows_v.at[b],
                out_hbm.at[pl.ds(base + c * chunk, chunk)],
                ssem.at[b],
            )
            cp.start()
            return cp

        def wait_scatter(c, b):
            pltpu.make_async_copy(
                rows_v.at[b],
                out_hbm.at[pl.ds(base + c * chunk, chunk)],
                ssem.at[b],
            ).wait()

        for c in range(_NBUF):
            gather(c, c)
        scatter(0, 0)

        n_steady = n_chunks - _NBUF
        assert n_steady % _NBUF == 0

        def body(j):
            c0 = 1 + j * _NBUF
            for u in range(_NBUF):
                c = c0 + u
                scatter(c, (1 + u) % _NBUF)
                wait_scatter(c - 1, u % _NBUF)
                gather(c + 3, u % _NBUF)

        pl.loop(0, n_steady // _NBUF)(body)

        for c in range(n_chunks - 3, n_chunks):
            scatter(c, c % _NBUF)
            wait_scatter(c - 1, (c - 1) % _NBUF)
        wait_scatter(n_chunks - 1, (n_chunks - 1) % _NBUF)

    return k


def kernel(x, pe):
    b, s = x.shape
    v, d = pe.shape
    n = b * s
    info = plsc.get_sparse_core_info()
    nw = info.num_cores * info.num_subcores
    b_per_w = n // nw
    chunk = 16
    n_chunks = b_per_w // chunk
    idx3 = x.astype(jnp.int32).reshape(nw, n_chunks, chunk)
    k = _gather_kernel(n, d, b_per_w, chunk, n_chunks)
    out = k(pe, idx3)
    return out.reshape(b, s, d)
